# Initial kernel scaffold; baseline (speedup 1.0000x reference)
import jax
import jax.numpy as jnp
from jax.experimental import pallas as pl


def kernel(feat_2d_all, sparse_feat_3d_F, links, coords_map_in, coords_map_out, W1, b1, g1, be1, W2, b2, g2, be2, W3, b3, g3, be3):
    raise NotImplementedError("write your pallas kernel here")


# speedup vs baseline: 44.9712x; 44.9712x over previous
import functools

import jax
import jax.numpy as jnp
from jax import lax
from jax.experimental import pallas as pl
from jax.experimental.pallas import tpu as pltpu
from jax.experimental.pallas import tpu_sc as plsc

_V = 3
_C2D = 64
_D3 = 96
_BS = 2
_H = 120
_W = 160
_N = 100000
_HW = _H * _W
_PAD_ROWS = 1920
_SLAB = _HW + _PAD_ROWS
_ZROW = _HW
_NVB = _V * _BS
_TROWS = _NVB * _SLAB

_NW = 32
_K = 256
_NBLK = 13
_NPAD = _NW * _K * _NBLK

_BN = 2000
_NTB = _N // _BN
_EPS = 1e-5


_TBLK = 1920


def _transpose_body(in_ref, out_ref):
    j = pl.program_id(1)

    @pl.when(j < 10)
    def _():
        out_ref[0] = in_ref[0].T

    @pl.when(j == 10)
    def _():
        out_ref[0] = jnp.zeros((_TBLK, _C2D), jnp.float32)


def _build_table(feat6):
    return pl.pallas_call(
        _transpose_body,
        grid=(_NVB, _SLAB // _TBLK),
        in_specs=[pl.BlockSpec((1, _C2D, _TBLK),
                               lambda i, j: (i, 0, jnp.minimum(j, 9)))],
        out_specs=pl.BlockSpec((1, _TBLK, _C2D), lambda i, j: (i, j, 0)),
        out_shape=jax.ShapeDtypeStruct((_NVB, _SLAB, _C2D), jnp.float32),
    )(feat6)


def _sc_body(table_hbm, rowids_hbm, cin_hbm, cout_hbm,
             x0_hbm, x1_hbm, x2_hbm,
             cin_v, cout_v, link_v, idx0, idx1, idx2, f0, f1, f2,
             gsem, ssem):
    wid = lax.axis_index("s") * 2 + lax.axis_index("c")

    def block(b, carry):
        gb = wid + _NW * b
        row = gb * 2
        pltpu.sync_copy(cin_hbm.at[pl.ds(row, 2)], cin_v)
        pltpu.sync_copy(cout_hbm.at[pl.ds(row, 2)], cout_v)
        for sub in range(2):
            pltpu.async_copy(rowids_hbm.at[cin_v.at[sub]],
                             link_v.at[pl.ds(sub * 128, 128)], gsem).wait()

        for sub in range(2):
            for g in range(8):
                base = sub * 128 + g * 16
                rows = lax.iota(jnp.int32, 16) + base
                for v, idx_ref in ((0, idx0), (1, idx1), (2, idx2)):
                    cv = jnp.full((16,), v, jnp.int32)
                    idx_ref[sub, pl.ds(g * 16, 16)] = plsc.load_gather(
                        link_v, [rows, cv])

        copies = []
        for idx_ref, f_ref in ((idx0, f0), (idx1, f1), (idx2, f2)):
            for sub in range(2):
                copies.append(pltpu.async_copy(
                    table_hbm.at[idx_ref.at[sub]],
                    f_ref.at[pl.ds(sub * 128, 128)], gsem))
        scats = []
        for (c0, c1), f_ref, x_hbm in zip(
                zip(copies[0::2], copies[1::2]), (f0, f1, f2),
                (x0_hbm, x1_hbm, x2_hbm)):
            c0.wait()
            c1.wait()
            for sub in range(2):
                scats.append(pltpu.async_copy(
                    f_ref.at[pl.ds(sub * 128, 128)],
                    x_hbm.at[cout_v.at[sub]], ssem))
        for s in scats:
            s.wait()
        return carry

    lax.fori_loop(0, _NBLK, block, 0)


def _rowid_body(lk_ref, out_ref):
    lk = lk_ref[...]
    cols = []
    for v in range(_V):
        b = lk[:, v:v + 1]
        vi = lk[:, 3 + v:4 + v]
        ui = lk[:, 6 + v:7 + v]
        val = lk[:, 9 + v:10 + v]
        pix = jnp.where(val > 0, vi * _W + ui, _ZROW)
        cols.append((v * _BS + b) * _SLAB + pix)
    cols.append(jnp.zeros((_BN, 16 - _V), jnp.int32))
    out_ref[...] = jnp.concatenate(cols, axis=1)


def _build_rowids(links2d):
    return pl.pallas_call(
        _rowid_body,
        grid=(_NTB,),
        in_specs=[pl.BlockSpec((_BN, 4 * _V), lambda i: (i, 0))],
        out_specs=pl.BlockSpec((_BN, 16), lambda i: (i, 0)),
        out_shape=jax.ShapeDtypeStruct((_N, 16), jnp.int32),
    )(links2d)


def _sc_gather(table2d, rowids, cin_p, cout_p):
    mesh = plsc.VectorSubcoreMesh(core_axis_name="c", subcore_axis_name="s",
                                  num_cores=2, num_subcores=16)
    xshape = jax.ShapeDtypeStruct((_NPAD, _C2D), jnp.float32)
    fn = pl.kernel(
        _sc_body,
        out_type=[xshape, xshape, xshape],
        mesh=mesh,
        compiler_params=pltpu.CompilerParams(needs_layout_passes=False,
                                             use_tc_tiling_on_sc=False),
        scratch_types=[
            pltpu.VMEM((2, 128), jnp.int32),
            pltpu.VMEM((2, 128), jnp.int32),
            pltpu.VMEM((_K, 16), jnp.int32),
            pltpu.VMEM((2, 128), jnp.int32),
            pltpu.VMEM((2, 128), jnp.int32),
            pltpu.VMEM((2, 128), jnp.int32),
            pltpu.VMEM((_K, _C2D), jnp.float32),
            pltpu.VMEM((_K, _C2D), jnp.float32),
            pltpu.VMEM((_K, _C2D), jnp.float32),
            pltpu.SemaphoreType.DMA,
            pltpu.SemaphoreType.DMA,
        ],
    )
    return fn(table2d, rowids, cin_p, cout_p)


def _scale_shift(stats_ref, g_ref, be_ref):
    inv_n = jnp.float32(1.0 / _N)
    mu = stats_ref[0:1, :] * inv_n
    var = stats_ref[1:2, :] * inv_n - mu * mu
    sc = g_ref[...] * lax.rsqrt(var + _EPS)
    sh = be_ref[...] - mu * sc
    return sc, sh


def _accum(i, y, acc_ref, stats_ref):
    @pl.when(i == 0)
    def _():
        acc_ref[...] = jnp.zeros_like(acc_ref)

    acc_ref[0:1, :] += jnp.sum(y, axis=0, keepdims=True)
    acc_ref[1:2, :] += jnp.sum(y * y, axis=0, keepdims=True)

    @pl.when(i == _NTB - 1)
    def _():
        stats_ref[...] = acc_ref[...]


def _p1_body(x0, x1, x2, w1, b1, y1_out, stats_out, acc):
    i = pl.program_id(0)
    y = (jnp.dot(x0[...], w1[0:64, :], preferred_element_type=jnp.float32)
         + jnp.dot(x1[...], w1[64:128, :], preferred_element_type=jnp.float32)
         + jnp.dot(x2[...], w1[128:192, :], preferred_element_type=jnp.float32)
         + b1[...])
    y1_out[...] = y
    _accum(i, y, acc, stats_out)


def _h1(y1_ref, s1, g1, be1):
    sc1, sh1 = _scale_shift(s1, g1, be1)
    return jnp.maximum(y1_ref[...] * sc1 + sh1, 0.0)


def _p2_body(y1, s1, g1, be1, w2, b2, stats_out, acc):
    i = pl.program_id(0)
    h = _h1(y1, s1, g1, be1)
    y2 = jnp.dot(h, w2[...], preferred_element_type=jnp.float32) + b2[...]
    _accum(i, y2, acc, stats_out)


def _y3(y1, f3d, s1, g1, be1, w2, b2, s2, g2, be2, w3, b3):
    h = _h1(y1, s1, g1, be1)
    y2 = jnp.dot(h, w2[...], preferred_element_type=jnp.float32) + b2[...]
    sc2, sh2 = _scale_shift(s2, g2, be2)
    h2 = jnp.maximum(y2 * sc2 + sh2, 0.0)
    return (jnp.dot(f3d[...], w3[0:_D3, :], preferred_element_type=jnp.float32)
            + jnp.dot(h2, w3[_D3:2 * _D3, :], preferred_element_type=jnp.float32)
            + b3[...])


def _p3_body(y1, f3d, s1, g1, be1, w2, b2, s2, g2, be2, w3, b3,
             stats_out, acc):
    i = pl.program_id(0)
    y3 = _y3(y1, f3d, s1, g1, be1, w2, b2, s2, g2, be2, w3, b3)
    _accum(i, y3, acc, stats_out)


def _p4_body(y1, f3d, s1, g1, be1, w2, b2, s2, g2, be2, w3, b3, s3, g3, be3,
             out):
    y3 = _y3(y1, f3d, s1, g1, be1, w2, b2, s2, g2, be2, w3, b3)
    sc3, sh3 = _scale_shift(s3, g3, be3)
    out[...] = jnp.maximum(y3 * sc3 + sh3, 0.0)


def _row_spec(c):
    return pl.BlockSpec((_BN, c), lambda i: (i, 0))


def _full_spec(r, c):
    return pl.BlockSpec((r, c), lambda i: (0, 0))


def _stats_spec(c):
    return pl.BlockSpec((2, c), lambda i: (0, 0))


def kernel(feat_2d_all, sparse_feat_3d_F, links, coords_map_in,
           coords_map_out, W1, b1, g1, be1, W2, b2, g2, be2, W3, b3, g3,
           be3):
    feat6 = feat_2d_all.reshape(_NVB, _C2D, _HW)
    table = _build_table(feat6).reshape(_TROWS, _C2D)

    links2d = links.reshape(_N, 4 * _V)
    pad = _NPAD - _N
    cin_p = jnp.concatenate(
        [coords_map_in, jnp.zeros((pad,), jnp.int32)]).reshape(-1, 128)
    cout_p = jnp.concatenate(
        [coords_map_out,
         jnp.arange(_N, _NPAD, dtype=jnp.int32)]).reshape(-1, 128)

    rowids = _build_rowids(links2d)
    x0, x1, x2 = _sc_gather(table, rowids, cin_p, cout_p)

    r = lambda a: a.reshape(1, -1)
    b1r, g1r, be1r = r(b1), r(g1), r(be1)
    b2r, g2r, be2r = r(b2), r(g2), r(be2)
    b3r, g3r, be3r = r(b3), r(g3), r(be3)

    y1, s1 = pl.pallas_call(
        _p1_body,
        grid=(_NTB,),
        in_specs=[_row_spec(_C2D), _row_spec(_C2D), _row_spec(_C2D),
                  _full_spec(_V * _C2D, _C2D), _full_spec(1, _C2D)],
        out_specs=[_row_spec(_C2D), _stats_spec(_C2D)],
        out_shape=[jax.ShapeDtypeStruct((_N, _C2D), jnp.float32),
                   jax.ShapeDtypeStruct((2, _C2D), jnp.float32)],
        scratch_shapes=[pltpu.VMEM((2, _C2D), jnp.float32)],
    )(x0, x1, x2, W1, b1r)

    s2 = pl.pallas_call(
        _p2_body,
        grid=(_NTB,),
        in_specs=[_row_spec(_C2D), _stats_spec(_C2D), _full_spec(1, _C2D),
                  _full_spec(1, _C2D), _full_spec(_C2D, _D3),
                  _full_spec(1, _D3)],
        out_specs=_stats_spec(_D3),
        out_shape=jax.ShapeDtypeStruct((2, _D3), jnp.float32),
        scratch_shapes=[pltpu.VMEM((2, _D3), jnp.float32)],
    )(y1, s1, g1r, be1r, W2, b2r)

    big_in = [_row_spec(_C2D), _row_spec(_D3), _stats_spec(_C2D),
              _full_spec(1, _C2D), _full_spec(1, _C2D),
              _full_spec(_C2D, _D3), _full_spec(1, _D3), _stats_spec(_D3),
              _full_spec(1, _D3), _full_spec(1, _D3),
              _full_spec(2 * _D3, _D3), _full_spec(1, _D3)]

    s3 = pl.pallas_call(
        _p3_body,
        grid=(_NTB,),
        in_specs=big_in,
        out_specs=_stats_spec(_D3),
        out_shape=jax.ShapeDtypeStruct((2, _D3), jnp.float32),
        scratch_shapes=[pltpu.VMEM((2, _D3), jnp.float32)],
    )(y1, sparse_feat_3d_F, s1, g1r, be1r, W2, b2r, s2, g2r, be2r, W3, b3r)

    out = pl.pallas_call(
        _p4_body,
        grid=(_NTB,),
        in_specs=big_in + [_stats_spec(_D3), _full_spec(1, _D3),
                           _full_spec(1, _D3)],
        out_specs=_row_spec(_D3),
        out_shape=jax.ShapeDtypeStruct((_N, _D3), jnp.float32),
    )(y1, sparse_feat_3d_F, s1, g1r, be1r, W2, b2r, s2, g2r, be2r, W3, b3r,
      s3, g3r, be3r)

    return out
